# baseline (speedup 1.0000x reference)
import functools

import jax
import jax.numpy as jnp
from jax import lax
from jax.experimental import pallas as pl
from jax.experimental.pallas import tpu as pltpu
from jax.experimental.pallas import tpu_sc as plsc

BATCH = 16384
EMB_DIM = 64
NUM_CORES = 2
NUM_SUBCORES = 16
NUM_WORKERS = NUM_CORES * NUM_SUBCORES
B_PER_W = BATCH // NUM_WORKERS
IDX_CHUNK = 128
N_CHUNKS = B_PER_W // IDX_CHUNK
GROUP = 16
N_GROUPS = B_PER_W // GROUP


def _ranker_body(x1_hbm, x2_hbm, ue_hbm, ce_hbm, out_hbm,
                 idx1_v, idx2_v, u_v, c_v, out_v, sem):
    wid = lax.axis_index("s") * NUM_CORES + lax.axis_index("c")
    base = wid * B_PER_W

    idx_copies = []
    for j in range(N_CHUNKS):
        off = base + j * IDX_CHUNK
        idx_copies.append(pltpu.async_copy(
            x1_hbm.at[pl.ds(off, IDX_CHUNK)], idx1_v.at[j], sem))
        idx_copies.append(pltpu.async_copy(
            x2_hbm.at[pl.ds(off, IDX_CHUNK)], idx2_v.at[j], sem))
    for cp in idx_copies:
        cp.wait()

    row_copies = []
    for j in range(N_CHUNKS):
        dst = pl.ds(j * IDX_CHUNK, IDX_CHUNK)
        row_copies.append(pltpu.async_copy(
            ue_hbm.at[idx1_v.at[j]], u_v.at[dst], sem))
        row_copies.append(pltpu.async_copy(
            ce_hbm.at[idx2_v.at[j]], c_v.at[dst], sem))
    for cp in row_copies:
        cp.wait()

    lane = lax.iota(jnp.int32, GROUP)

    def group_body(g, carry):
        rbase = g * GROUP
        res = jnp.zeros((GROUP,), jnp.float32)
        for r in range(GROUP):
            row = rbase + r
            acc = u_v[row, pl.ds(0, 16)] * c_v[row, pl.ds(0, 16)]
            for k in range(1, EMB_DIM // 16):
                acc = acc + u_v[row, pl.ds(k * 16, 16)] * c_v[row, pl.ds(k * 16, 16)]
            res = jnp.where(lane == r, jnp.sum(acc), res)
        out_v[pl.ds(rbase, GROUP)] = 1.0 / (1.0 + jnp.exp(-res))
        return carry

    lax.fori_loop(0, N_GROUPS, group_body, 0, unroll=False)

    pltpu.sync_copy(out_v, out_hbm.at[pl.ds(base, B_PER_W)])


@jax.jit
def _ranker(x1, x2, ue, ce):
    mesh = plsc.VectorSubcoreMesh(core_axis_name="c", subcore_axis_name="s")
    return pl.kernel(
        _ranker_body,
        out_type=jax.ShapeDtypeStruct((BATCH,), jnp.float32),
        mesh=mesh,
        scratch_types=[
            pltpu.VMEM((N_CHUNKS, IDX_CHUNK), jnp.int32),
            pltpu.VMEM((N_CHUNKS, IDX_CHUNK), jnp.int32),
            pltpu.VMEM((B_PER_W, EMB_DIM), jnp.float32),
            pltpu.VMEM((B_PER_W, EMB_DIM), jnp.float32),
            pltpu.VMEM((B_PER_W,), jnp.float32),
            pltpu.SemaphoreType.DMA,
        ],
        compiler_params=pltpu.CompilerParams(
            needs_layout_passes=False, use_tc_tiling_on_sc=False),
    )(x1, x2, ue, ce)


def kernel(x1, x2, Ue, Ce, W, b):
    del W, b
    return _ranker(x1, x2, Ue, Ce)


# speedup vs baseline: 1.5532x; 1.5532x over previous
import jax
import jax.numpy as jnp
from jax import lax
from jax.experimental import pallas as pl
from jax.experimental.pallas import tpu as pltpu
from jax.experimental.pallas import tpu_sc as plsc

BATCH = 16384
EMB_DIM = 64
NUM_CORES = 2
NUM_SUBCORES = 16
NUM_WORKERS = NUM_CORES * NUM_SUBCORES
B_PER_W = BATCH // NUM_WORKERS
GROUP = 16
N_GROUPS = B_PER_W // GROUP
DEPTH = 4


def _ranker_body(x1_hbm, x2_hbm, ue_hbm, ce_hbm, out_hbm,
                 idx1_v, idx2_v, u_b, c_b, out_v,
                 sem_i, sem0, sem1, sem2, sem3):
    sems = (sem0, sem1, sem2, sem3)
    wid = lax.axis_index("s") * NUM_CORES + lax.axis_index("c")
    base = wid * B_PER_W

    cp1 = pltpu.async_copy(x1_hbm.at[pl.ds(base, B_PER_W)], idx1_v, sem_i)
    cp2 = pltpu.async_copy(x2_hbm.at[pl.ds(base, B_PER_W)], idx2_v, sem_i)
    cp1.wait()
    cp2.wait()

    lane = lax.iota(jnp.int32, GROUP)

    def fire(g, s):
        iv1 = idx1_v[pl.ds(g * GROUP, GROUP)]
        iv2 = idx2_v[pl.ds(g * GROUP, GROUP)]
        for r in range(GROUP):
            pltpu.async_copy(ue_hbm.at[iv1[r]], u_b.at[s, r], sems[s])
            pltpu.async_copy(ce_hbm.at[iv2[r]], c_b.at[s, r], sems[s])

    for s in range(DEPTH):
        fire(s, s)

    def iter_body(i, carry):
        for s in range(DEPTH):
            g = i * DEPTH + s
            for r in range(GROUP):
                pltpu.make_async_copy(ue_hbm.at[0], u_b.at[s, r], sems[s]).wait()
                pltpu.make_async_copy(ce_hbm.at[0], c_b.at[s, r], sems[s]).wait()
            res = jnp.zeros((GROUP,), jnp.float32)
            for r in range(GROUP):
                acc = u_b[s, r, pl.ds(0, 16)] * c_b[s, r, pl.ds(0, 16)]
                for k in range(1, EMB_DIM // 16):
                    acc = acc + (u_b[s, r, pl.ds(k * 16, 16)]
                                 * c_b[s, r, pl.ds(k * 16, 16)])
                res = jnp.where(lane == r, jnp.sum(acc), res)
            out_v[pl.ds(g * GROUP, GROUP)] = 1.0 / (1.0 + jnp.exp(-res))

            @pl.when(g + DEPTH < N_GROUPS)
            def _():
                fire(g + DEPTH, s)
        return carry

    lax.fori_loop(0, N_GROUPS // DEPTH, iter_body, 0, unroll=False)

    pltpu.sync_copy(out_v, out_hbm.at[pl.ds(base, B_PER_W)])


@jax.jit
def _ranker(x1, x2, ue, ce):
    mesh = plsc.VectorSubcoreMesh(core_axis_name="c", subcore_axis_name="s")
    return pl.kernel(
        _ranker_body,
        out_type=jax.ShapeDtypeStruct((BATCH,), jnp.float32),
        mesh=mesh,
        scratch_types=[
            pltpu.VMEM((B_PER_W,), jnp.int32),
            pltpu.VMEM((B_PER_W,), jnp.int32),
            pltpu.VMEM((DEPTH, GROUP, EMB_DIM), jnp.float32),
            pltpu.VMEM((DEPTH, GROUP, EMB_DIM), jnp.float32),
            pltpu.VMEM((B_PER_W,), jnp.float32),
            pltpu.SemaphoreType.DMA,
            pltpu.SemaphoreType.DMA,
            pltpu.SemaphoreType.DMA,
            pltpu.SemaphoreType.DMA,
            pltpu.SemaphoreType.DMA,
        ],
        compiler_params=pltpu.CompilerParams(needs_layout_passes=False),
    )(x1, x2, ue, ce)


def kernel(x1, x2, Ue, Ce, W, b):
    del W, b
    return _ranker(x1, x2, Ue, Ce)


# speedup vs baseline: 1.5549x; 1.0011x over previous
import jax
import jax.numpy as jnp
from jax import lax
from jax.experimental import pallas as pl
from jax.experimental.pallas import tpu as pltpu
from jax.experimental.pallas import tpu_sc as plsc

BATCH = 16384
EMB_DIM = 64
NUM_CORES = 2
NUM_SUBCORES = 16
NUM_WORKERS = NUM_CORES * NUM_SUBCORES
B_PER_W = BATCH // NUM_WORKERS
GROUP = 16
N_GROUPS = B_PER_W // GROUP
DEPTH = 4


def _ranker_body(x1_hbm, x2_hbm, ue_hbm, ce_hbm, out_hbm,
                 idx1_v, idx2_v, u_b, c_b, out_v,
                 sem_i, sem0, sem1, sem2, sem3, sem4, sem5, sem6, sem7):
    sems = (sem0, sem1, sem2, sem3, sem4, sem5, sem6, sem7)
    wid = lax.axis_index("s") * NUM_CORES + lax.axis_index("c")
    base = wid * B_PER_W

    cp1 = pltpu.async_copy(x1_hbm.at[pl.ds(base, B_PER_W)], idx1_v, sem_i)
    cp2 = pltpu.async_copy(x2_hbm.at[pl.ds(base, B_PER_W)], idx2_v, sem_i)
    cp1.wait()
    cp2.wait()

    lane = lax.iota(jnp.int32, GROUP)

    def fire(g, s):
        iv1 = idx1_v[pl.ds(g * GROUP, GROUP)]
        iv2 = idx2_v[pl.ds(g * GROUP, GROUP)]
        for r in range(GROUP):
            sem = sems[2 * s + (r & 1)]
            pltpu.async_copy(ue_hbm.at[iv1[r]], u_b.at[s, r], sem)
            pltpu.async_copy(ce_hbm.at[iv2[r]], c_b.at[s, r], sem)

    for s in range(DEPTH):
        fire(s, s)

    def iter_body(i, carry):
        for s in range(DEPTH):
            g = i * DEPTH + s
            for r in range(GROUP):
                sem = sems[2 * s + (r & 1)]
                pltpu.make_async_copy(ue_hbm.at[0], u_b.at[s, r], sem).wait()
                pltpu.make_async_copy(ce_hbm.at[0], c_b.at[s, r], sem).wait()
            res = jnp.zeros((GROUP,), jnp.float32)
            for r in range(GROUP):
                acc = u_b[s, r, pl.ds(0, 16)] * c_b[s, r, pl.ds(0, 16)]
                for k in range(1, EMB_DIM // 16):
                    acc = acc + (u_b[s, r, pl.ds(k * 16, 16)]
                                 * c_b[s, r, pl.ds(k * 16, 16)])
                res = jnp.where(lane == r, jnp.sum(acc), res)
            out_v[pl.ds(g * GROUP, GROUP)] = 1.0 / (1.0 + jnp.exp(-res))

            @pl.when(g + DEPTH < N_GROUPS)
            def _():
                fire(g + DEPTH, s)
        return carry

    lax.fori_loop(0, N_GROUPS // DEPTH, iter_body, 0, unroll=False)

    pltpu.sync_copy(out_v, out_hbm.at[pl.ds(base, B_PER_W)])


@jax.jit
def _ranker(x1, x2, ue, ce):
    mesh = plsc.VectorSubcoreMesh(core_axis_name="c", subcore_axis_name="s")
    return pl.kernel(
        _ranker_body,
        out_type=jax.ShapeDtypeStruct((BATCH,), jnp.float32),
        mesh=mesh,
        scratch_types=[
            pltpu.VMEM((B_PER_W,), jnp.int32),
            pltpu.VMEM((B_PER_W,), jnp.int32),
            pltpu.VMEM((DEPTH, GROUP, EMB_DIM), jnp.float32),
            pltpu.VMEM((DEPTH, GROUP, EMB_DIM), jnp.float32),
            pltpu.VMEM((B_PER_W,), jnp.float32),
        ] + [pltpu.SemaphoreType.DMA] * 9,
        compiler_params=pltpu.CompilerParams(needs_layout_passes=False),
    )(x1, x2, ue, ce)


def kernel(x1, x2, Ue, Ce, W, b):
    del W, b
    return _ranker(x1, x2, Ue, Ce)
